# baseline (speedup 1.0000x reference)
import functools

import jax
import jax.numpy as jnp
from jax import lax
from jax.experimental import pallas as pl
from jax.experimental.pallas import tpu as pltpu
from jax.experimental.pallas import tpu_sc as plsc

_NE = 1024
_D = 64
_BR = 1024
_NROWS = 32 * 1024


def _tc_body(xt_ref, e_ref, idx_ref, loss_ref):
    i = pl.program_id(0)
    xt = xt_ref[...]
    ssq = jnp.sum(xt * xt, axis=0, keepdims=True)
    norm = jnp.sqrt(ssq)
    inv = 1.0 / jnp.maximum(norm, 1e-12)
    xnt = xt * inv

    e = e_ref[...]
    esq = jnp.sum(e * e, axis=1, keepdims=True)
    s = lax.dot_general(e, xnt, (((1,), (0,)), ((), ())),
                        preferred_element_type=jnp.float32)
    g = esq - 2.0 * s
    m = jnp.min(g, axis=0, keepdims=True)
    row_ids = lax.broadcasted_iota(jnp.int32, g.shape, 0)
    idx = jnp.min(jnp.where(g == m, row_ids, _NE), axis=0, keepdims=True)
    idx_ref[0] = idx

    xnsq = ssq * (inv * inv)
    part = jnp.sum(xnsq + m) * (1.0 / (_NROWS * _D))

    @pl.when(i == 0)
    def _():
        loss_ref[0, 0] = 0.0

    loss_ref[0, 0] += part


_tc_call = pl.pallas_call(
    _tc_body,
    grid=(_NROWS // _BR,),
    in_specs=[
        pl.BlockSpec((_D, _BR), lambda i: (0, i)),
        pl.BlockSpec((_NE, _D), lambda i: (0, 0)),
    ],
    out_specs=[
        pl.BlockSpec((1, 1, _BR), lambda i: (i, 0, 0)),
        pl.BlockSpec(memory_space=pltpu.SMEM, block_shape=(1, 1),
                     index_map=lambda i: (0, 0)),
    ],
    out_shape=[
        jax.ShapeDtypeStruct((_NROWS // _BR, 1, _BR), jnp.int32),
        jax.ShapeDtypeStruct((1, 1), jnp.float32),
    ],
    compiler_params=pltpu.CompilerParams(dimension_semantics=("arbitrary",)),
)


_NC, _NS = 2, 16
_NW = _NC * _NS
_BPW = _NROWS // _NW


@functools.cache
def _sc_gather_call():
    @functools.partial(
        pl.kernel,
        mesh=plsc.VectorSubcoreMesh(core_axis_name="c", subcore_axis_name="s"),
        out_type=jax.ShapeDtypeStruct((_NROWS, _D), jnp.float32),
        scratch_types=[
            pltpu.VMEM((_BPW,), jnp.int32),
            pltpu.VMEM((_BPW, _D), jnp.float32),
            pltpu.SemaphoreType.DMA,
        ],
        compiler_params=pltpu.CompilerParams(use_tc_tiling_on_sc=False),
    )
    def _sc_gather(table_hbm, idx_hbm, out_hbm, idx_v, rows_v, sem):
        wid = lax.axis_index("s") * _NC + lax.axis_index("c")
        base = wid * _BPW
        pltpu.sync_copy(idx_hbm.at[pl.ds(base, _BPW)], idx_v)
        pltpu.async_copy(table_hbm.at[idx_v], rows_v, sem).wait()
        pltpu.sync_copy(rows_v, out_hbm.at[pl.ds(base, _BPW)])

    return _sc_gather


def kernel(inputs, embeddings):
    orig_shape = inputs.shape
    x2d = inputs.reshape(-1, _D)
    xt = x2d.T
    idx2d, loss11 = _tc_call(xt, embeddings)
    q = _sc_gather_call()(embeddings, idx2d.reshape(-1))
    quantized = q.reshape(orig_shape)
    loss = loss11[0, 0]
    encoding_indices = idx2d.reshape(orig_shape[:-1])
    return (quantized, loss, encoding_indices)
